# baseline (speedup 1.0000x reference)
import numpy as np
import jax
import jax.numpy as jnp
from jax.experimental import pallas as pl
from jax.experimental.pallas import tpu as pltpu

_NUM_ROWS = 100000
_D = 256
_ORD_LEN = 64
_OFFSETS = (0, 137, 271)
_STRIDE = 7919
_N_OUT = len(_OFFSETS) * _ORD_LEN
_SPLIT = 152

_IDX = [((np.arange(_ORD_LEN, dtype=np.int64) * _STRIDE + off) % _NUM_ROWS)
        .astype(int).tolist() for off in _OFFSETS]
_FLAT = [(l, int(r)) for l in range(len(_OFFSETS)) for r in _IDX[l]]


def _tc_body(l0, l1, l2, out_ref, stage, sem_a, sem_b):
    refs = (l0, l1, l2)
    for p in range(_SPLIT):
        l, row = _FLAT[p]
        pltpu.make_async_copy(refs[l].at[pl.ds(row, 1)],
                              stage.at[pl.ds(p, 1)], sem_a).start()
    for p in range(_SPLIT, _N_OUT):
        l, row = _FLAT[p]
        pltpu.make_async_copy(refs[l].at[pl.ds(row, 1)],
                              out_ref.at[pl.ds(p, 1)], sem_b).start()
    pltpu.make_async_copy(l0.at[pl.ds(0, _SPLIT)], stage, sem_a).wait()
    fwd = pltpu.make_async_copy(stage, out_ref.at[pl.ds(0, _SPLIT)], sem_a)
    fwd.start()
    pltpu.make_async_copy(l0.at[pl.ds(0, _N_OUT - _SPLIT)],
                          out_ref.at[pl.ds(_SPLIT, _N_OUT - _SPLIT)],
                          sem_b).wait()
    fwd.wait()


def kernel(layer_0, layer_1, layer_2):
    return pl.pallas_call(
        _tc_body,
        out_shape=jax.ShapeDtypeStruct((_N_OUT, _D), jnp.float32),
        in_specs=[pl.BlockSpec(memory_space=pltpu.MemorySpace.HBM)] * 3,
        out_specs=pl.BlockSpec(memory_space=pltpu.MemorySpace.HBM),
        scratch_shapes=[
            pltpu.VMEM((_SPLIT, _D), jnp.float32),
            pltpu.SemaphoreType.DMA,
            pltpu.SemaphoreType.DMA,
        ],
    )(layer_0, layer_1, layer_2)


# speedup vs baseline: 1.3063x; 1.3063x over previous
import numpy as np
import jax
import jax.numpy as jnp
from jax.experimental import pallas as pl
from jax.experimental.pallas import tpu as pltpu

_NUM_ROWS = 100000
_D = 256
_ORD_LEN = 64
_OFFSETS = (0, 137, 271)
_STRIDE = 7919

_IDX = [((np.arange(_ORD_LEN, dtype=np.int64) * _STRIDE + off) % _NUM_ROWS)
        .astype(int).tolist() for off in _OFFSETS]


def _tc_body(l0, l1, l2, out_ref, sem):
    for l, ref in enumerate((l0, l1, l2)):
        for i, row in enumerate(_IDX[l]):
            pltpu.make_async_copy(
                ref.at[pl.ds(row, 1)],
                out_ref.at[pl.ds(l * _ORD_LEN + i, 1)],
                sem).start()
    pltpu.make_async_copy(l0.at[pl.ds(0, len(_OFFSETS) * _ORD_LEN)],
                          out_ref, sem).wait()


def kernel(layer_0, layer_1, layer_2):
    return pl.pallas_call(
        _tc_body,
        out_shape=jax.ShapeDtypeStruct((len(_OFFSETS) * _ORD_LEN, _D),
                                       jnp.float32),
        in_specs=[pl.BlockSpec(memory_space=pltpu.MemorySpace.HBM)] * 3,
        out_specs=pl.BlockSpec((len(_OFFSETS) * _ORD_LEN, _D),
                               lambda: (0, 0)),
        scratch_shapes=[pltpu.SemaphoreType.DMA],
    )(layer_0, layer_1, layer_2)
